# baseline (speedup 1.0000x reference)
import jax
import jax.numpy as jnp
from jax import lax
from jax.experimental import pallas as pl
from jax.experimental.pallas import tpu as pltpu
from jax.experimental.pallas import tpu_sc as plsc

N = 10000
D = 128
H = 128

NC = 2
NS = 16
NW = NC * NS
CHUNK = 128
NP = 10240
RPT = NP // NS
DEC_UNROLL = 16
DEC_A = 48
DEC_LIGHT_CORE = 0

_mesh = plsc.VectorSubcoreMesh(
    core_axis_name="c", subcore_axis_name="s", num_cores=NC, num_subcores=NS)


def _worker(c, s):
    return c * NS + s


def _deg_body(dst3, degp, dstbuf, acc):
    c = lax.axis_index("c")
    s = lax.axis_index("s")
    w = _worker(c, s)
    nch = dst3.shape[1]

    def fz(i, _):
        acc[pl.ds(i * 16, 16)] = jnp.zeros((16,), jnp.float32)
        return 0
    lax.fori_loop(0, NP // 16, fz, 0)

    pltpu.sync_copy(dst3.at[w], dstbuf)
    ones16 = jnp.ones((16,), jnp.float32)

    def f(j, _):
        iv = dstbuf[j // 8, pl.ds((j % 8) * 16, 16)]
        plsc.addupdate_scatter(acc, [iv], ones16)
        return 0
    lax.fori_loop(0, nch * 8, f, 0)

    pltpu.sync_copy(acc, degp.at[w])


def _make_deg(nch):
    return pl.kernel(
        _deg_body,
        out_type=jax.ShapeDtypeStruct((NW, NP), jnp.float32),
        mesh=_mesh,
        scratch_types=[
            pltpu.VMEM((nch, CHUNK), jnp.int32),
            pltpu.VMEM((NP,), jnp.float32),
        ],
        compiler_params=pltpu.CompilerParams(needs_layout_passes=False),
    )


def _agg_body(y, src3, dst3, aggp, srcbuf, dstbuf, rows, rows2, sem0, sem1,
              accsh):
    c = lax.axis_index("c")
    s = lax.axis_index("s")
    w = _worker(c, s)

    def fz(t, _):
        rows[t // 8, pl.ds((t % 8) * 16, 16)] = jnp.zeros((16,), jnp.float32)
        return 0
    lax.fori_loop(0, CHUNK * 8, fz, 0)

    for k in range(RPT // CHUNK):
        pltpu.sync_copy(rows, accsh.at[pl.ds(s * RPT + k * CHUNK, CHUNK)])
    plsc.subcore_barrier()

    nch = src3.shape[1]
    hch = nch // 2
    bufs = (rows, rows2)
    sems = (sem0, sem1)

    def g_start(j, b):
        pltpu.make_async_copy(y.at[srcbuf.at[j]], bufs[b], sems[b]).start()

    def g_wait(j, b):
        pltpu.make_async_copy(y.at[srcbuf.at[j]], bufs[b], sems[b]).wait()

    for h in range(2):
        pltpu.sync_copy(src3.at[w, pl.ds(h * hch, hch)], srcbuf)
        pltpu.sync_copy(dst3.at[w, pl.ds(h * hch, hch)], dstbuf)
        g_start(0, 0)

        def pair(p, _):
            j0 = 2 * p
            g_start(j0 + 1, 1)
            g_wait(j0, 0)
            pltpu.sync_copy(rows, accsh.at[dstbuf.at[j0]], add=True)

            @pl.when(p + 1 < hch // 2)
            def _():
                g_start(j0 + 2, 0)
            g_wait(j0 + 1, 1)
            pltpu.sync_copy(rows2, accsh.at[dstbuf.at[j0 + 1]], add=True)
            return 0
        lax.fori_loop(0, hch // 2, pair, 0)

    plsc.subcore_barrier()
    for k in range(RPT // CHUNK):
        sl = pl.ds(s * RPT + k * CHUNK, CHUNK)
        pltpu.sync_copy(accsh.at[sl], aggp.at[c, sl])


def _make_agg(nch):
    assert nch % 4 == 0
    return pl.kernel(
        _agg_body,
        out_type=jax.ShapeDtypeStruct((NC, NP, D), jnp.float32),
        mesh=_mesh,
        scratch_types=[
            pltpu.VMEM((nch // 2, CHUNK), jnp.int32),
            pltpu.VMEM((nch // 2, CHUNK), jnp.int32),
            pltpu.VMEM((CHUNK, D), jnp.float32),
            pltpu.VMEM((CHUNK, D), jnp.float32),
            pltpu.SemaphoreType.DMA,
            pltpu.SemaphoreType.DMA,
            pltpu.VMEM_SHARED((NP, D), jnp.float32),
        ],
    )


def _dec_body(z, e03, e13, out, e0b, e1b, a0, b0, a1, b1,
              sa0, sb0, sa1, sb1, ob):
    c = lax.axis_index("c")
    s = lax.axis_index("s")
    nch = e03.shape[1]
    epw = nch * CHUNK

    abufs = (a0, a1)
    bbufs = (b0, b1)
    asems = (sa0, sa1)
    bsems = (sb0, sb1)

    def g_start(j, p):
        pltpu.make_async_copy(z.at[e0b.at[j]], abufs[p], asems[p]).start()
        pltpu.make_async_copy(z.at[e1b.at[j]], bbufs[p], bsems[p]).start()

    def g_wait(j, p):
        pltpu.make_async_copy(z.at[e0b.at[j]], abufs[p], asems[p]).wait()
        pltpu.make_async_copy(z.at[e1b.at[j]], bbufs[p], bsems[p]).wait()

    def compute(j, ab, bb):
        base = j * CHUNK

        def edges(q, _):
            for u in range(4):
                e = q * 4 + u
                ps = []
                for k in range(D // 16):
                    av = ab[e, pl.ds(k * 16, 16)]
                    bv = bb[e, pl.ds(k * 16, 16)]
                    ps.append(av * bv)
                while len(ps) > 1:
                    ps = [ps[i] + ps[i + 1] for i in range(0, len(ps), 2)]
                plsc.addupdate_scatter(
                    ob, [jnp.full((16,), base + e, jnp.int32)], ps[0])
            return 0
        lax.fori_loop(0, CHUNK // 4, edges, 0)

    def run(nchunks):
        def fz(i, _):
            ob[pl.ds(i * 16, 16)] = jnp.zeros((16,), jnp.float32)
            return 0
        lax.fori_loop(0, nchunks * CHUNK // 16, fz, 0)
        g_start(0, 0)

        def pair(p, _):
            j0 = 2 * p
            g_start(j0 + 1, 1)
            g_wait(j0, 0)
            compute(j0, a0, b0)

            @pl.when(p + 1 < nchunks // 2)
            def _():
                g_start(j0 + 2, 0)
            g_wait(j0 + 1, 1)
            compute(j0 + 1, a1, b1)
            return 0
        lax.fori_loop(0, nchunks // 2, pair, 0)

    rem = nch - DEC_A
    lrow = DEC_LIGHT_CORE * NS + s
    hrow = (1 - DEC_LIGHT_CORE) * NS + s

    @pl.when(c == DEC_LIGHT_CORE)
    def _():
        pltpu.sync_copy(e03.at[lrow, pl.ds(0, DEC_A)], e0b.at[pl.ds(0, DEC_A)])
        pltpu.sync_copy(e13.at[lrow, pl.ds(0, DEC_A)], e1b.at[pl.ds(0, DEC_A)])
        run(DEC_A)
        pltpu.sync_copy(ob.at[pl.ds(0, DEC_A * CHUNK)],
                        out.at[pl.ds(lrow * epw, DEC_A * CHUNK)])

    @pl.when(c != DEC_LIGHT_CORE)
    def _():
        pltpu.sync_copy(e03.at[hrow], e0b.at[pl.ds(0, nch)])
        pltpu.sync_copy(e03.at[lrow, pl.ds(DEC_A, rem)],
                        e0b.at[pl.ds(nch, rem)])
        pltpu.sync_copy(e13.at[hrow], e1b.at[pl.ds(0, nch)])
        pltpu.sync_copy(e13.at[lrow, pl.ds(DEC_A, rem)],
                        e1b.at[pl.ds(nch, rem)])
        run(nch + rem)
        pltpu.sync_copy(ob.at[pl.ds(0, nch * CHUNK)],
                        out.at[pl.ds(hrow * epw, nch * CHUNK)])
        pltpu.sync_copy(ob.at[pl.ds(nch * CHUNK, rem * CHUNK)],
                        out.at[pl.ds(lrow * epw + DEC_A * CHUNK, rem * CHUNK)])


def _make_dec(nch):
    assert nch % 2 == 0 and DEC_A % 2 == 0 and (2 * nch - DEC_A) % 2 == 0
    return pl.kernel(
        _dec_body,
        out_type=jax.ShapeDtypeStruct((NW * nch * CHUNK,), jnp.float32),
        mesh=_mesh,
        scratch_types=[
            pltpu.VMEM((2 * nch - DEC_A, CHUNK), jnp.int32),
            pltpu.VMEM((2 * nch - DEC_A, CHUNK), jnp.int32),
            pltpu.VMEM((CHUNK, D), jnp.float32),
            pltpu.VMEM((CHUNK, D), jnp.float32),
            pltpu.VMEM((CHUNK, D), jnp.float32),
            pltpu.VMEM((CHUNK, D), jnp.float32),
            pltpu.SemaphoreType.DMA,
            pltpu.SemaphoreType.DMA,
            pltpu.SemaphoreType.DMA,
            pltpu.SemaphoreType.DMA,
            pltpu.VMEM(((2 * nch - DEC_A) * CHUNK,), jnp.float32),
        ],
        compiler_params=pltpu.CompilerParams(needs_layout_passes=False),
    )


BLK = 512


def _dinv_blk(degp_blk):
    ones = jnp.ones((NW, D), jnp.float32)
    deg = lax.dot_general(degp_blk, ones, (((0,), (0,)), ((), ())),
                          preferred_element_type=jnp.float32)
    return lax.rsqrt(deg + 1.0)


def _tc1_body(x_ref, w_ref, degp_ref, y_ref):
    dinv = _dinv_blk(degp_ref[...])
    y_ref[...] = jnp.dot(x_ref[...], w_ref[...],
                         preferred_element_type=jnp.float32) * dinv


def _tc2_body(y1_ref, aggp_ref, degp_ref, b_ref, w_ref, y2_ref):
    dinv = _dinv_blk(degp_ref[...])
    agg = aggp_ref[0] + aggp_ref[1]
    h = jnp.maximum(dinv * (agg + y1_ref[...]) + b_ref[...], 0.0)
    y2_ref[...] = jnp.dot(h, w_ref[...],
                          preferred_element_type=jnp.float32) * dinv


def _tc3_body(y2_ref, aggp_ref, degp_ref, b_ref, z_ref):
    dinv = _dinv_blk(degp_ref[...])
    agg = aggp_ref[0] + aggp_ref[1]
    z_ref[...] = dinv * (agg + y2_ref[...]) + b_ref[...]


_deg_spec = pl.BlockSpec((NW, BLK), lambda i: (0, i))
_agg_spec = pl.BlockSpec((2, BLK, D), lambda i: (0, i, 0))
_row_spec = pl.BlockSpec((BLK, D), lambda i: (i, 0))
_w_spec = pl.BlockSpec((D, H), lambda i: (0, 0))
_b_spec = pl.BlockSpec((1, H), lambda i: (0, 0))

_tc1 = pl.pallas_call(
    _tc1_body,
    grid=(NP // BLK,),
    in_specs=[_row_spec, _w_spec, _deg_spec],
    out_specs=_row_spec,
    out_shape=jax.ShapeDtypeStruct((NP, H), jnp.float32),
)

_tc2 = pl.pallas_call(
    _tc2_body,
    grid=(NP // BLK,),
    in_specs=[_row_spec, _agg_spec, _deg_spec, _b_spec, _w_spec],
    out_specs=_row_spec,
    out_shape=jax.ShapeDtypeStruct((NP, H), jnp.float32),
)

_tc3 = pl.pallas_call(
    _tc3_body,
    grid=(NP // BLK,),
    in_specs=[_row_spec, _agg_spec, _deg_spec, _b_spec],
    out_specs=_row_spec,
    out_shape=jax.ShapeDtypeStruct((NP, H), jnp.float32),
)


def kernel(x, edge_index, edge_label_index, W1, b1, W2, b2):
    src, dst = edge_index[0], edge_index[1]
    e0, e1 = edge_label_index[0], edge_label_index[1]
    E = src.shape[0]
    EL = e0.shape[0]

    quant = NW * CHUNK * 4
    pe = (-E) % quant
    pel = (-EL) % quant
    src3 = jnp.concatenate(
        [src, jnp.zeros((pe,), jnp.int32)]).reshape(NW, -1, CHUNK)
    dst3 = jnp.concatenate(
        [dst, jnp.full((pe,), N, jnp.int32)]).reshape(NW, -1, CHUNK)
    e03 = jnp.concatenate(
        [e0, jnp.zeros((pel,), jnp.int32)]).reshape(NW, -1, CHUNK)
    e13 = jnp.concatenate(
        [e1, jnp.zeros((pel,), jnp.int32)]).reshape(NW, -1, CHUNK)
    b1r = b1.reshape(1, H)
    b2r = b2.reshape(1, H)
    xp = jnp.concatenate([x, jnp.zeros((NP - x.shape[0], D), x.dtype)])

    degp = _make_deg(dst3.shape[1])(dst3)

    agg_fn = _make_agg(src3.shape[1])
    y1 = _tc1(xp, W1, degp)
    aggp1 = agg_fn(y1, src3, dst3)
    y2 = _tc2(y1, aggp1, degp, b1r, W2)
    aggp2 = agg_fn(y2, src3, dst3)
    z = _tc3(y2, aggp2, degp, b2r)

    dec = _make_dec(e03.shape[1])(z, e03, e13)
    return dec[:EL]


# speedup vs baseline: 1.0944x; 1.0944x over previous
import jax
import jax.numpy as jnp
from jax import lax
from jax.experimental import pallas as pl
from jax.experimental.pallas import tpu as pltpu
from jax.experimental.pallas import tpu_sc as plsc

N = 10000
D = 128
H = 128

NC = 2
NS = 16
NW = NC * NS
CHUNK = 128
NP = 10240
RPT = NP // NS
DEC_UNROLL = 16
DEC_A = 48
DEC_LIGHT_CORE = 1

_mesh = plsc.VectorSubcoreMesh(
    core_axis_name="c", subcore_axis_name="s", num_cores=NC, num_subcores=NS)


def _worker(c, s):
    return c * NS + s


def _deg_body(dst3, degp, dstbuf, acc):
    c = lax.axis_index("c")
    s = lax.axis_index("s")
    w = _worker(c, s)
    nch = dst3.shape[1]

    def fz(i, _):
        acc[pl.ds(i * 16, 16)] = jnp.zeros((16,), jnp.float32)
        return 0
    lax.fori_loop(0, NP // 16, fz, 0)

    pltpu.sync_copy(dst3.at[w], dstbuf)
    ones16 = jnp.ones((16,), jnp.float32)

    def f(j, _):
        iv = dstbuf[j // 8, pl.ds((j % 8) * 16, 16)]
        plsc.addupdate_scatter(acc, [iv], ones16)
        return 0
    lax.fori_loop(0, nch * 8, f, 0)

    pltpu.sync_copy(acc, degp.at[w])


def _make_deg(nch):
    return pl.kernel(
        _deg_body,
        out_type=jax.ShapeDtypeStruct((NW, NP), jnp.float32),
        mesh=_mesh,
        scratch_types=[
            pltpu.VMEM((nch, CHUNK), jnp.int32),
            pltpu.VMEM((NP,), jnp.float32),
        ],
        compiler_params=pltpu.CompilerParams(needs_layout_passes=False),
    )


def _agg_body(y, src3, dst3, aggp, srcbuf, dstbuf, rows, rows2, sem0, sem1,
              accsh):
    c = lax.axis_index("c")
    s = lax.axis_index("s")
    w = _worker(c, s)

    def fz(t, _):
        rows[t // 8, pl.ds((t % 8) * 16, 16)] = jnp.zeros((16,), jnp.float32)
        return 0
    lax.fori_loop(0, CHUNK * 8, fz, 0)

    for k in range(RPT // CHUNK):
        pltpu.sync_copy(rows, accsh.at[pl.ds(s * RPT + k * CHUNK, CHUNK)])
    plsc.subcore_barrier()

    nch = src3.shape[1]
    hch = nch // 2
    bufs = (rows, rows2)
    sems = (sem0, sem1)

    def g_start(j, b):
        pltpu.make_async_copy(y.at[srcbuf.at[j]], bufs[b], sems[b]).start()

    def g_wait(j, b):
        pltpu.make_async_copy(y.at[srcbuf.at[j]], bufs[b], sems[b]).wait()

    for h in range(2):
        pltpu.sync_copy(src3.at[w, pl.ds(h * hch, hch)], srcbuf)
        pltpu.sync_copy(dst3.at[w, pl.ds(h * hch, hch)], dstbuf)
        g_start(0, 0)

        def pair(p, _):
            j0 = 2 * p
            g_start(j0 + 1, 1)
            g_wait(j0, 0)
            pltpu.sync_copy(rows, accsh.at[dstbuf.at[j0]], add=True)

            @pl.when(p + 1 < hch // 2)
            def _():
                g_start(j0 + 2, 0)
            g_wait(j0 + 1, 1)
            pltpu.sync_copy(rows2, accsh.at[dstbuf.at[j0 + 1]], add=True)
            return 0
        lax.fori_loop(0, hch // 2, pair, 0)

    plsc.subcore_barrier()
    for k in range(RPT // CHUNK):
        sl = pl.ds(s * RPT + k * CHUNK, CHUNK)
        pltpu.sync_copy(accsh.at[sl], aggp.at[c, sl])


def _make_agg(nch):
    assert nch % 4 == 0
    return pl.kernel(
        _agg_body,
        out_type=jax.ShapeDtypeStruct((NC, NP, D), jnp.float32),
        mesh=_mesh,
        scratch_types=[
            pltpu.VMEM((nch // 2, CHUNK), jnp.int32),
            pltpu.VMEM((nch // 2, CHUNK), jnp.int32),
            pltpu.VMEM((CHUNK, D), jnp.float32),
            pltpu.VMEM((CHUNK, D), jnp.float32),
            pltpu.SemaphoreType.DMA,
            pltpu.SemaphoreType.DMA,
            pltpu.VMEM_SHARED((NP, D), jnp.float32),
        ],
    )


def _dec_body(z, e03, e13, out, e0b, e1b, a0, b0, a1, b1,
              sa0, sb0, sa1, sb1, ob):
    c = lax.axis_index("c")
    s = lax.axis_index("s")
    nch = e03.shape[1]
    epw = nch * CHUNK

    abufs = (a0, a1)
    bbufs = (b0, b1)
    asems = (sa0, sa1)
    bsems = (sb0, sb1)

    def g_start(j, p):
        pltpu.make_async_copy(z.at[e0b.at[j]], abufs[p], asems[p]).start()
        pltpu.make_async_copy(z.at[e1b.at[j]], bbufs[p], bsems[p]).start()

    def g_wait(j, p):
        pltpu.make_async_copy(z.at[e0b.at[j]], abufs[p], asems[p]).wait()
        pltpu.make_async_copy(z.at[e1b.at[j]], bbufs[p], bsems[p]).wait()

    def compute(j, ab, bb):
        base = j * CHUNK

        def edges(q, _):
            for u in range(4):
                e = q * 4 + u
                ps = []
                for k in range(D // 16):
                    av = ab[e, pl.ds(k * 16, 16)]
                    bv = bb[e, pl.ds(k * 16, 16)]
                    ps.append(av * bv)
                while len(ps) > 1:
                    ps = [ps[i] + ps[i + 1] for i in range(0, len(ps), 2)]
                plsc.addupdate_scatter(
                    ob, [jnp.full((16,), base + e, jnp.int32)], ps[0])
            return 0
        lax.fori_loop(0, CHUNK // 4, edges, 0)

    def run(nchunks):
        def fz(i, _):
            ob[pl.ds(i * 16, 16)] = jnp.zeros((16,), jnp.float32)
            return 0
        lax.fori_loop(0, nchunks * CHUNK // 16, fz, 0)
        g_start(0, 0)

        def pair(p, _):
            j0 = 2 * p
            g_start(j0 + 1, 1)
            g_wait(j0, 0)
            compute(j0, a0, b0)

            @pl.when(p + 1 < nchunks // 2)
            def _():
                g_start(j0 + 2, 0)
            g_wait(j0 + 1, 1)
            compute(j0 + 1, a1, b1)
            return 0
        lax.fori_loop(0, nchunks // 2, pair, 0)

    rem = nch - DEC_A
    lrow = DEC_LIGHT_CORE * NS + s
    hrow = (1 - DEC_LIGHT_CORE) * NS + s

    @pl.when(c == DEC_LIGHT_CORE)
    def _():
        pltpu.sync_copy(e03.at[lrow, pl.ds(0, DEC_A)], e0b.at[pl.ds(0, DEC_A)])
        pltpu.sync_copy(e13.at[lrow, pl.ds(0, DEC_A)], e1b.at[pl.ds(0, DEC_A)])
        run(DEC_A)
        pltpu.sync_copy(ob.at[pl.ds(0, DEC_A * CHUNK)],
                        out.at[pl.ds(lrow * epw, DEC_A * CHUNK)])

    @pl.when(c != DEC_LIGHT_CORE)
    def _():
        pltpu.sync_copy(e03.at[hrow], e0b.at[pl.ds(0, nch)])
        pltpu.sync_copy(e03.at[lrow, pl.ds(DEC_A, rem)],
                        e0b.at[pl.ds(nch, rem)])
        pltpu.sync_copy(e13.at[hrow], e1b.at[pl.ds(0, nch)])
        pltpu.sync_copy(e13.at[lrow, pl.ds(DEC_A, rem)],
                        e1b.at[pl.ds(nch, rem)])
        run(nch + rem)
        pltpu.sync_copy(ob.at[pl.ds(0, nch * CHUNK)],
                        out.at[pl.ds(hrow * epw, nch * CHUNK)])
        pltpu.sync_copy(ob.at[pl.ds(nch * CHUNK, rem * CHUNK)],
                        out.at[pl.ds(lrow * epw + DEC_A * CHUNK, rem * CHUNK)])


def _make_dec(nch):
    assert nch % 2 == 0 and DEC_A % 2 == 0 and (2 * nch - DEC_A) % 2 == 0
    return pl.kernel(
        _dec_body,
        out_type=jax.ShapeDtypeStruct((NW * nch * CHUNK,), jnp.float32),
        mesh=_mesh,
        scratch_types=[
            pltpu.VMEM((2 * nch - DEC_A, CHUNK), jnp.int32),
            pltpu.VMEM((2 * nch - DEC_A, CHUNK), jnp.int32),
            pltpu.VMEM((CHUNK, D), jnp.float32),
            pltpu.VMEM((CHUNK, D), jnp.float32),
            pltpu.VMEM((CHUNK, D), jnp.float32),
            pltpu.VMEM((CHUNK, D), jnp.float32),
            pltpu.SemaphoreType.DMA,
            pltpu.SemaphoreType.DMA,
            pltpu.SemaphoreType.DMA,
            pltpu.SemaphoreType.DMA,
            pltpu.VMEM(((2 * nch - DEC_A) * CHUNK,), jnp.float32),
        ],
        compiler_params=pltpu.CompilerParams(needs_layout_passes=False),
    )


BLK = 512


def _dinv_blk(degp_blk):
    ones = jnp.ones((NW, D), jnp.float32)
    deg = lax.dot_general(degp_blk, ones, (((0,), (0,)), ((), ())),
                          preferred_element_type=jnp.float32)
    return lax.rsqrt(deg + 1.0)


def _tc1_body(x_ref, w_ref, degp_ref, y_ref):
    dinv = _dinv_blk(degp_ref[...])
    y_ref[...] = jnp.dot(x_ref[...], w_ref[...],
                         preferred_element_type=jnp.float32) * dinv


def _tc2_body(y1_ref, aggp_ref, degp_ref, b_ref, w_ref, y2_ref):
    dinv = _dinv_blk(degp_ref[...])
    agg = aggp_ref[0] + aggp_ref[1]
    h = jnp.maximum(dinv * (agg + y1_ref[...]) + b_ref[...], 0.0)
    y2_ref[...] = jnp.dot(h, w_ref[...],
                          preferred_element_type=jnp.float32) * dinv


def _tc3_body(y2_ref, aggp_ref, degp_ref, b_ref, z_ref):
    dinv = _dinv_blk(degp_ref[...])
    agg = aggp_ref[0] + aggp_ref[1]
    z_ref[...] = dinv * (agg + y2_ref[...]) + b_ref[...]


_deg_spec = pl.BlockSpec((NW, BLK), lambda i: (0, i))
_agg_spec = pl.BlockSpec((2, BLK, D), lambda i: (0, i, 0))
_row_spec = pl.BlockSpec((BLK, D), lambda i: (i, 0))
_w_spec = pl.BlockSpec((D, H), lambda i: (0, 0))
_b_spec = pl.BlockSpec((1, H), lambda i: (0, 0))

_tc1 = pl.pallas_call(
    _tc1_body,
    grid=(NP // BLK,),
    in_specs=[_row_spec, _w_spec, _deg_spec],
    out_specs=_row_spec,
    out_shape=jax.ShapeDtypeStruct((NP, H), jnp.float32),
)

_tc2 = pl.pallas_call(
    _tc2_body,
    grid=(NP // BLK,),
    in_specs=[_row_spec, _agg_spec, _deg_spec, _b_spec, _w_spec],
    out_specs=_row_spec,
    out_shape=jax.ShapeDtypeStruct((NP, H), jnp.float32),
)

_tc3 = pl.pallas_call(
    _tc3_body,
    grid=(NP // BLK,),
    in_specs=[_row_spec, _agg_spec, _deg_spec, _b_spec],
    out_specs=_row_spec,
    out_shape=jax.ShapeDtypeStruct((NP, H), jnp.float32),
)


def kernel(x, edge_index, edge_label_index, W1, b1, W2, b2):
    src, dst = edge_index[0], edge_index[1]
    e0, e1 = edge_label_index[0], edge_label_index[1]
    E = src.shape[0]
    EL = e0.shape[0]

    quant = NW * CHUNK * 4
    pe = (-E) % quant
    pel = (-EL) % quant
    src3 = jnp.concatenate(
        [src, jnp.zeros((pe,), jnp.int32)]).reshape(NW, -1, CHUNK)
    dst3 = jnp.concatenate(
        [dst, jnp.full((pe,), N, jnp.int32)]).reshape(NW, -1, CHUNK)
    e03 = jnp.concatenate(
        [e0, jnp.zeros((pel,), jnp.int32)]).reshape(NW, -1, CHUNK)
    e13 = jnp.concatenate(
        [e1, jnp.zeros((pel,), jnp.int32)]).reshape(NW, -1, CHUNK)
    b1r = b1.reshape(1, H)
    b2r = b2.reshape(1, H)
    xp = jnp.concatenate([x, jnp.zeros((NP - x.shape[0], D), x.dtype)])

    degp = _make_deg(dst3.shape[1])(dst3)

    agg_fn = _make_agg(src3.shape[1])
    y1 = _tc1(xp, W1, degp)
    aggp1 = agg_fn(y1, src3, dst3)
    y2 = _tc2(y1, aggp1, degp, b1r, W2)
    aggp2 = agg_fn(y2, src3, dst3)
    z = _tc3(y2, aggp2, degp, b2r)

    dec = _make_dec(e03.shape[1])(z, e03, e13)
    return dec[:EL]


# speedup vs baseline: 1.1056x; 1.0102x over previous
import jax
import jax.numpy as jnp
from jax import lax
from jax.experimental import pallas as pl
from jax.experimental.pallas import tpu as pltpu
from jax.experimental.pallas import tpu_sc as plsc

N = 10000
D = 128
H = 128

NC = 2
NS = 16
NW = NC * NS
CHUNK = 128
NP = 10240
RPT = NP // NS
DEC_UNROLL = 16
DEC_A = 48
DEC_LIGHT_CORE = 1

_mesh = plsc.VectorSubcoreMesh(
    core_axis_name="c", subcore_axis_name="s", num_cores=NC, num_subcores=NS)


def _worker(c, s):
    return c * NS + s


def _deg_body(dst3, degp, dstbuf, acc):
    c = lax.axis_index("c")
    s = lax.axis_index("s")
    w = _worker(c, s)
    nch = dst3.shape[1]

    def fz(i, _):
        acc[pl.ds(i * 16, 16)] = jnp.zeros((16,), jnp.float32)
        return 0
    lax.fori_loop(0, NP // 16, fz, 0)

    pltpu.sync_copy(dst3.at[w], dstbuf)
    ones16 = jnp.ones((16,), jnp.float32)

    def f(j, _):
        iv = dstbuf[j // 8, pl.ds((j % 8) * 16, 16)]
        plsc.addupdate_scatter(acc, [iv], ones16)
        return 0
    lax.fori_loop(0, nch * 8, f, 0)

    pltpu.sync_copy(acc, degp.at[w])


def _make_deg(nch):
    return pl.kernel(
        _deg_body,
        out_type=jax.ShapeDtypeStruct((NW, NP), jnp.float32),
        mesh=_mesh,
        scratch_types=[
            pltpu.VMEM((nch, CHUNK), jnp.int32),
            pltpu.VMEM((NP,), jnp.float32),
        ],
        compiler_params=pltpu.CompilerParams(needs_layout_passes=False),
    )


def _agg_body(y, src3, dst3, aggp, srcbuf, dstbuf, rows, rows2, sem0, sem1,
              accsh):
    c = lax.axis_index("c")
    s = lax.axis_index("s")
    w = _worker(c, s)

    def fz(t, _):
        rows[t // 8, pl.ds((t % 8) * 16, 16)] = jnp.zeros((16,), jnp.float32)
        return 0
    lax.fori_loop(0, CHUNK * 8, fz, 0)

    for k in range(RPT // CHUNK):
        pltpu.sync_copy(rows, accsh.at[pl.ds(s * RPT + k * CHUNK, CHUNK)])
    plsc.subcore_barrier()

    nch = src3.shape[1]
    hch = nch // 2
    bufs = (rows, rows2)
    sems = (sem0, sem1)

    def g_start(j, b):
        pltpu.make_async_copy(y.at[srcbuf.at[j]], bufs[b], sems[b]).start()

    def g_wait(j, b):
        pltpu.make_async_copy(y.at[srcbuf.at[j]], bufs[b], sems[b]).wait()

    def seg(row, start):
        pltpu.sync_copy(src3.at[row, pl.ds(start, hch)], srcbuf)
        pltpu.sync_copy(dst3.at[row, pl.ds(start, hch)], dstbuf)
        g_start(0, 0)

        def pair(p, _):
            j0 = 2 * p
            g_start(j0 + 1, 1)
            g_wait(j0, 0)
            pltpu.sync_copy(rows, accsh.at[dstbuf.at[j0]], add=True)

            @pl.when(p + 1 < hch // 2)
            def _():
                g_start(j0 + 2, 0)
            g_wait(j0 + 1, 1)
            pltpu.sync_copy(rows2, accsh.at[dstbuf.at[j0 + 1]], add=True)
            return 0
        lax.fori_loop(0, hch // 2, pair, 0)

    lrow = DEC_LIGHT_CORE * NS + s
    hrow = (1 - DEC_LIGHT_CORE) * NS + s

    @pl.when(c == DEC_LIGHT_CORE)
    def _():
        seg(lrow, 0)

    @pl.when(c != DEC_LIGHT_CORE)
    def _():
        seg(hrow, 0)
        seg(hrow, hch)
        seg(lrow, hch)

    plsc.subcore_barrier()
    for k in range(RPT // CHUNK):
        sl = pl.ds(s * RPT + k * CHUNK, CHUNK)
        pltpu.sync_copy(accsh.at[sl], aggp.at[c, sl])


def _make_agg(nch):
    assert nch % 4 == 0
    return pl.kernel(
        _agg_body,
        out_type=jax.ShapeDtypeStruct((NC, NP, D), jnp.float32),
        mesh=_mesh,
        scratch_types=[
            pltpu.VMEM((nch // 2, CHUNK), jnp.int32),
            pltpu.VMEM((nch // 2, CHUNK), jnp.int32),
            pltpu.VMEM((CHUNK, D), jnp.float32),
            pltpu.VMEM((CHUNK, D), jnp.float32),
            pltpu.SemaphoreType.DMA,
            pltpu.SemaphoreType.DMA,
            pltpu.VMEM_SHARED((NP, D), jnp.float32),
        ],
    )


def _dec_body(z, e03, e13, out, e0b, e1b, a0, b0, a1, b1,
              sa0, sb0, sa1, sb1, ob):
    c = lax.axis_index("c")
    s = lax.axis_index("s")
    nch = e03.shape[1]
    epw = nch * CHUNK

    abufs = (a0, a1)
    bbufs = (b0, b1)
    asems = (sa0, sa1)
    bsems = (sb0, sb1)

    def g_start(j, p):
        pltpu.make_async_copy(z.at[e0b.at[j]], abufs[p], asems[p]).start()
        pltpu.make_async_copy(z.at[e1b.at[j]], bbufs[p], bsems[p]).start()

    def g_wait(j, p):
        pltpu.make_async_copy(z.at[e0b.at[j]], abufs[p], asems[p]).wait()
        pltpu.make_async_copy(z.at[e1b.at[j]], bbufs[p], bsems[p]).wait()

    def compute(j, ab, bb):
        base = j * CHUNK

        def edges(q, _):
            for u in range(4):
                e = q * 4 + u
                ps = []
                for k in range(D // 16):
                    av = ab[e, pl.ds(k * 16, 16)]
                    bv = bb[e, pl.ds(k * 16, 16)]
                    ps.append(av * bv)
                while len(ps) > 1:
                    ps = [ps[i] + ps[i + 1] for i in range(0, len(ps), 2)]
                plsc.addupdate_scatter(
                    ob, [jnp.full((16,), base + e, jnp.int32)], ps[0])
            return 0
        lax.fori_loop(0, CHUNK // 4, edges, 0)

    def run(nchunks):
        def fz(i, _):
            ob[pl.ds(i * 16, 16)] = jnp.zeros((16,), jnp.float32)
            return 0
        lax.fori_loop(0, nchunks * CHUNK // 16, fz, 0)
        g_start(0, 0)

        def pair(p, _):
            j0 = 2 * p
            g_start(j0 + 1, 1)
            g_wait(j0, 0)
            compute(j0, a0, b0)

            @pl.when(p + 1 < nchunks // 2)
            def _():
                g_start(j0 + 2, 0)
            g_wait(j0 + 1, 1)
            compute(j0 + 1, a1, b1)
            return 0
        lax.fori_loop(0, nchunks // 2, pair, 0)

    rem = nch - DEC_A
    lrow = DEC_LIGHT_CORE * NS + s
    hrow = (1 - DEC_LIGHT_CORE) * NS + s

    @pl.when(c == DEC_LIGHT_CORE)
    def _():
        pltpu.sync_copy(e03.at[lrow, pl.ds(0, DEC_A)], e0b.at[pl.ds(0, DEC_A)])
        pltpu.sync_copy(e13.at[lrow, pl.ds(0, DEC_A)], e1b.at[pl.ds(0, DEC_A)])
        run(DEC_A)
        pltpu.sync_copy(ob.at[pl.ds(0, DEC_A * CHUNK)],
                        out.at[pl.ds(lrow * epw, DEC_A * CHUNK)])

    @pl.when(c != DEC_LIGHT_CORE)
    def _():
        pltpu.sync_copy(e03.at[hrow], e0b.at[pl.ds(0, nch)])
        pltpu.sync_copy(e03.at[lrow, pl.ds(DEC_A, rem)],
                        e0b.at[pl.ds(nch, rem)])
        pltpu.sync_copy(e13.at[hrow], e1b.at[pl.ds(0, nch)])
        pltpu.sync_copy(e13.at[lrow, pl.ds(DEC_A, rem)],
                        e1b.at[pl.ds(nch, rem)])
        run(nch + rem)
        pltpu.sync_copy(ob.at[pl.ds(0, nch * CHUNK)],
                        out.at[pl.ds(hrow * epw, nch * CHUNK)])
        pltpu.sync_copy(ob.at[pl.ds(nch * CHUNK, rem * CHUNK)],
                        out.at[pl.ds(lrow * epw + DEC_A * CHUNK, rem * CHUNK)])


def _make_dec(nch):
    assert nch % 2 == 0 and DEC_A % 2 == 0 and (2 * nch - DEC_A) % 2 == 0
    return pl.kernel(
        _dec_body,
        out_type=jax.ShapeDtypeStruct((NW * nch * CHUNK,), jnp.float32),
        mesh=_mesh,
        scratch_types=[
            pltpu.VMEM((2 * nch - DEC_A, CHUNK), jnp.int32),
            pltpu.VMEM((2 * nch - DEC_A, CHUNK), jnp.int32),
            pltpu.VMEM((CHUNK, D), jnp.float32),
            pltpu.VMEM((CHUNK, D), jnp.float32),
            pltpu.VMEM((CHUNK, D), jnp.float32),
            pltpu.VMEM((CHUNK, D), jnp.float32),
            pltpu.SemaphoreType.DMA,
            pltpu.SemaphoreType.DMA,
            pltpu.SemaphoreType.DMA,
            pltpu.SemaphoreType.DMA,
            pltpu.VMEM(((2 * nch - DEC_A) * CHUNK,), jnp.float32),
        ],
        compiler_params=pltpu.CompilerParams(needs_layout_passes=False),
    )


BLK = 512


def _dinv_blk(degp_blk):
    ones = jnp.ones((NW, D), jnp.float32)
    deg = lax.dot_general(degp_blk, ones, (((0,), (0,)), ((), ())),
                          preferred_element_type=jnp.float32)
    return lax.rsqrt(deg + 1.0)


def _tc1_body(x_ref, w_ref, degp_ref, y_ref):
    dinv = _dinv_blk(degp_ref[...])
    y_ref[...] = jnp.dot(x_ref[...], w_ref[...],
                         preferred_element_type=jnp.float32) * dinv


def _tc2_body(y1_ref, aggp_ref, degp_ref, b_ref, w_ref, y2_ref):
    dinv = _dinv_blk(degp_ref[...])
    agg = aggp_ref[0] + aggp_ref[1]
    h = jnp.maximum(dinv * (agg + y1_ref[...]) + b_ref[...], 0.0)
    y2_ref[...] = jnp.dot(h, w_ref[...],
                          preferred_element_type=jnp.float32) * dinv


def _tc3_body(y2_ref, aggp_ref, degp_ref, b_ref, z_ref):
    dinv = _dinv_blk(degp_ref[...])
    agg = aggp_ref[0] + aggp_ref[1]
    z_ref[...] = dinv * (agg + y2_ref[...]) + b_ref[...]


_deg_spec = pl.BlockSpec((NW, BLK), lambda i: (0, i))
_agg_spec = pl.BlockSpec((2, BLK, D), lambda i: (0, i, 0))
_row_spec = pl.BlockSpec((BLK, D), lambda i: (i, 0))
_w_spec = pl.BlockSpec((D, H), lambda i: (0, 0))
_b_spec = pl.BlockSpec((1, H), lambda i: (0, 0))

_tc1 = pl.pallas_call(
    _tc1_body,
    grid=(NP // BLK,),
    in_specs=[_row_spec, _w_spec, _deg_spec],
    out_specs=_row_spec,
    out_shape=jax.ShapeDtypeStruct((NP, H), jnp.float32),
)

_tc2 = pl.pallas_call(
    _tc2_body,
    grid=(NP // BLK,),
    in_specs=[_row_spec, _agg_spec, _deg_spec, _b_spec, _w_spec],
    out_specs=_row_spec,
    out_shape=jax.ShapeDtypeStruct((NP, H), jnp.float32),
)

_tc3 = pl.pallas_call(
    _tc3_body,
    grid=(NP // BLK,),
    in_specs=[_row_spec, _agg_spec, _deg_spec, _b_spec],
    out_specs=_row_spec,
    out_shape=jax.ShapeDtypeStruct((NP, H), jnp.float32),
)


def kernel(x, edge_index, edge_label_index, W1, b1, W2, b2):
    src, dst = edge_index[0], edge_index[1]
    e0, e1 = edge_label_index[0], edge_label_index[1]
    E = src.shape[0]
    EL = e0.shape[0]

    quant = NW * CHUNK * 4
    pe = (-E) % quant
    pel = (-EL) % quant
    src3 = jnp.concatenate(
        [src, jnp.zeros((pe,), jnp.int32)]).reshape(NW, -1, CHUNK)
    dst3 = jnp.concatenate(
        [dst, jnp.full((pe,), N, jnp.int32)]).reshape(NW, -1, CHUNK)
    e03 = jnp.concatenate(
        [e0, jnp.zeros((pel,), jnp.int32)]).reshape(NW, -1, CHUNK)
    e13 = jnp.concatenate(
        [e1, jnp.zeros((pel,), jnp.int32)]).reshape(NW, -1, CHUNK)
    b1r = b1.reshape(1, H)
    b2r = b2.reshape(1, H)
    xp = jnp.concatenate([x, jnp.zeros((NP - x.shape[0], D), x.dtype)])

    degp = _make_deg(dst3.shape[1])(dst3)

    agg_fn = _make_agg(src3.shape[1])
    y1 = _tc1(xp, W1, degp)
    aggp1 = agg_fn(y1, src3, dst3)
    y2 = _tc2(y1, aggp1, degp, b1r, W2)
    aggp2 = agg_fn(y2, src3, dst3)
    z = _tc3(y2, aggp2, degp, b2r)

    dec = _make_dec(e03.shape[1])(z, e03, e13)
    return dec[:EL]
